# baseline (speedup 1.0000x reference)
import functools
import math

import jax
import jax.numpy as jnp
from jax.experimental import pallas as pl
from jax.experimental.pallas import tpu as pltpu

_WP = 32
_CHUNK = 128
_PAIR = 16
_NSETS = 4


def _one_image(xa, we_ref, wd_ref, bd_ref, wp_ref, bp_ref,
               pf0, pf1, pf2, hdw, *, C, W, hid, Sp):
    zb = jnp.zeros((40, hid), jnp.bfloat16)
    pf0[pl.ds(0, 40)] = zb
    pf1[pl.ds(0, 32)] = zb[:32]
    pf2[pl.ds(0, 32)] = zb[:32]
    pf0[pl.ds(Sp + 32, 32)] = zb[:32]
    pf1[pl.ds(Sp + 32, 32)] = zb[:32]
    pf2[pl.ds(Sp + 32, 32)] = zb[:32]

    mrow = (jax.lax.broadcasted_iota(jnp.int32, (1, _CHUNK), 1) % _WP
            < W).astype(jnp.bfloat16)

    for ci in range(Sp // _CHUNK):
        base = ci * _CHUNK
        xc = jnp.concatenate([xa[:, base:base + _CHUNK], mrow], axis=0)
        h = jax.lax.dot_general(
            xc, we_ref[...],
            (((0,), (0,)), ((), ())),
            preferred_element_type=jnp.float32)
        hbc = jnp.clip(h, 0.0, 6.0).astype(jnp.bfloat16)
        pf0[pl.ds(33 + base, _CHUNK)] = hbc
        pf1[pl.ds(32 + base, _CHUNK)] = hbc
        pf2[pl.ds(31 + base, _CHUNK)] = hbc

    pfs = (pf0, pf1, pf2)
    for ci in range(Sp // _CHUNK):
        base = ci * _CHUNK
        acc = None
        for dh in range(3):
            for dx in range(3):
                slab = pfs[dx][pl.ds(32 * dh + base, _CHUNK)]
                term = slab * wd_ref[3 * dh + dx, :]
                acc = term if acc is None else acc + term
        hdw[pl.ds(base, _CHUNK)] = jnp.clip(acc + bd_ref[...], 0.0, 6.0)

    y = jax.lax.dot_general(
        wp_ref[...], hdw[...],
        (((0,), (1,)), ((), ())),
        preferred_element_type=jnp.float32)
    y = y + bp_ref[...] + xa.astype(jnp.float32)
    return y.astype(jnp.bfloat16)


def _block_body(x_ref, we_ref, wd_ref, bd_ref, wp_ref, bp_ref, o_ref,
                *scratch, C, W, hid, Sp, pair):
    sets = [scratch[4 * i:4 * i + 4] for i in range(_NSETS)]
    args = (we_ref, wd_ref, bd_ref, wp_ref, bp_ref)
    kw = dict(C=C, W=W, hid=hid, Sp=Sp)
    for img in range(pair):
        o_ref[img] = _one_image(x_ref[img], *args, *sets[img % _NSETS], **kw)


def kernel(x_nchw, w_exp, b_exp, w_dw, b_dw, w_proj, b_proj):
    N, C, H, W = x_nchw.shape
    hid = w_exp.shape[1]
    Sp = H * _WP

    xb = x_nchw.astype(jnp.bfloat16)
    xp = jnp.pad(xb, ((0, 0), (0, 0), (0, 0), (0, _WP - W)))
    xa = xp.reshape(N, C, Sp)

    we = jnp.concatenate([w_exp, b_exp], axis=0).astype(jnp.bfloat16)
    wd = w_dw.astype(jnp.bfloat16)
    bd = b_dw.astype(jnp.bfloat16)
    wp = w_proj.astype(jnp.bfloat16)
    bp = jnp.transpose(b_proj)

    pair = math.gcd(N, _PAIR)
    body = functools.partial(_block_body, C=C, W=W, hid=hid, Sp=Sp, pair=pair)

    def full(shape):
        nd = len(shape)
        return pl.BlockSpec(shape, lambda n, nd=nd: (0,) * nd)

    scratch = []
    for _ in range(_NSETS):
        scratch += [pltpu.VMEM((Sp + 64, hid), jnp.bfloat16) for _ in range(3)]
        scratch.append(pltpu.VMEM((Sp, hid), jnp.bfloat16))

    out = pl.pallas_call(
        body,
        out_shape=jax.ShapeDtypeStruct((N, C, Sp), jnp.bfloat16),
        grid=(N // pair,),
        in_specs=[
            pl.BlockSpec((pair, C, Sp), lambda n: (n, 0, 0)),
            full(we.shape),
            full(wd.shape),
            full(bd.shape),
            full(wp.shape),
            full(bp.shape),
        ],
        out_specs=pl.BlockSpec((pair, C, Sp), lambda n: (n, 0, 0)),
        scratch_shapes=scratch,
        compiler_params=pltpu.CompilerParams(
            dimension_semantics=("parallel",)),
    )(xa, we, wd, bd, wp, bp)

    return out.reshape(N, C, H, _WP)[..., :W].astype(jnp.float32)


# speedup vs baseline: 1.0519x; 1.0519x over previous
import functools
import math

import jax
import jax.numpy as jnp
from jax.experimental import pallas as pl
from jax.experimental.pallas import tpu as pltpu

_WP = 32
_CHUNK = 128
_PAIR = 8
_NSETS = 4


def _one_image(xa, we_ref, wd_ref, bd_ref, wp_ref, bp_ref,
               pf0, pf1, pf2, hdw, *, C, W, hid, Sp):
    zb = jnp.zeros((40, hid), jnp.bfloat16)
    pf0[pl.ds(0, 40)] = zb
    pf1[pl.ds(0, 32)] = zb[:32]
    pf2[pl.ds(0, 32)] = zb[:32]
    pf0[pl.ds(Sp + 32, 32)] = zb[:32]
    pf1[pl.ds(Sp + 32, 32)] = zb[:32]
    pf2[pl.ds(Sp + 32, 32)] = zb[:32]

    mrow = (jax.lax.broadcasted_iota(jnp.int32, (1, _CHUNK), 1) % _WP
            < W).astype(jnp.bfloat16)

    for ci in range(Sp // _CHUNK):
        base = ci * _CHUNK
        xc = jnp.concatenate([xa[:, base:base + _CHUNK], mrow], axis=0)
        h = jax.lax.dot_general(
            xc, we_ref[...],
            (((0,), (0,)), ((), ())),
            preferred_element_type=jnp.float32)
        hbc = jnp.clip(h, 0.0, 6.0).astype(jnp.bfloat16)
        pf0[pl.ds(33 + base, _CHUNK)] = hbc
        pf1[pl.ds(32 + base, _CHUNK)] = hbc
        pf2[pl.ds(31 + base, _CHUNK)] = hbc

    pfs = (pf0, pf1, pf2)
    for ci in range(Sp // _CHUNK):
        base = ci * _CHUNK
        acc = None
        for dh in range(3):
            for dx in range(3):
                slab = pfs[dx][pl.ds(32 * dh + base, _CHUNK)]
                term = slab * wd_ref[3 * dh + dx, :]
                acc = term if acc is None else acc + term
        hdw[pl.ds(base, _CHUNK)] = jnp.clip(acc + bd_ref[...], 0.0, 6.0)

    y = jax.lax.dot_general(
        wp_ref[...], hdw[...],
        (((0,), (1,)), ((), ())),
        preferred_element_type=jnp.float32)
    y = y + bp_ref[...] + xa.astype(jnp.float32)
    return y.astype(jnp.bfloat16)


def _block_body(x_ref, we_ref, wd_ref, bd_ref, wp_ref, bp_ref, o_ref,
                *scratch, C, W, hid, Sp, pair):
    sets = [scratch[4 * i:4 * i + 4] for i in range(_NSETS)]
    args = (we_ref, wd_ref, bd_ref, wp_ref, bp_ref)
    kw = dict(C=C, W=W, hid=hid, Sp=Sp)
    for img in range(pair):
        o_ref[img] = _one_image(x_ref[img], *args, *sets[img % _NSETS], **kw)


def kernel(x_nchw, w_exp, b_exp, w_dw, b_dw, w_proj, b_proj):
    N, C, H, W = x_nchw.shape
    hid = w_exp.shape[1]
    Sp = H * _WP

    xb = x_nchw.astype(jnp.bfloat16)
    xp = jnp.pad(xb, ((0, 0), (0, 0), (0, 0), (0, _WP - W)))
    xa = xp.reshape(N, C, Sp)

    we = jnp.concatenate([w_exp, b_exp], axis=0).astype(jnp.bfloat16)
    wd = w_dw.astype(jnp.bfloat16)
    bd = b_dw.astype(jnp.bfloat16)
    wp = w_proj.astype(jnp.bfloat16)
    bp = jnp.transpose(b_proj)

    pair = math.gcd(N, _PAIR)
    body = functools.partial(_block_body, C=C, W=W, hid=hid, Sp=Sp, pair=pair)

    def full(shape):
        nd = len(shape)
        return pl.BlockSpec(shape, lambda n, nd=nd: (0,) * nd)

    scratch = []
    for _ in range(_NSETS):
        scratch += [pltpu.VMEM((Sp + 64, hid), jnp.bfloat16) for _ in range(3)]
        scratch.append(pltpu.VMEM((Sp, hid), jnp.bfloat16))

    out = pl.pallas_call(
        body,
        out_shape=jax.ShapeDtypeStruct((N, C, Sp), jnp.bfloat16),
        grid=(N // pair,),
        in_specs=[
            pl.BlockSpec((pair, C, Sp), lambda n: (n, 0, 0)),
            full(we.shape),
            full(wd.shape),
            full(bd.shape),
            full(wp.shape),
            full(bp.shape),
        ],
        out_specs=pl.BlockSpec((pair, C, Sp), lambda n: (n, 0, 0)),
        scratch_shapes=scratch,
        compiler_params=pltpu.CompilerParams(
            dimension_semantics=("parallel",)),
    )(xa, we, wd, bd, wp, bp)

    return out.reshape(N, C, H, _WP)[..., :W].astype(jnp.float32)


# speedup vs baseline: 1.0813x; 1.0279x over previous
import functools
import math

import jax
import jax.numpy as jnp
from jax.experimental import pallas as pl
from jax.experimental.pallas import tpu as pltpu

_WP = 32
_CHUNK = 128
_PAIR = 8
_NSETS = 2


def _one_image(xa, we_ref, wd_ref, bd_ref, wp_ref, bp_ref,
               pf0, pf1, pf2, hdw, *, C, W, hid, Sp):
    zb = jnp.zeros((40, hid), jnp.bfloat16)
    pf0[pl.ds(0, 40)] = zb
    pf1[pl.ds(0, 32)] = zb[:32]
    pf2[pl.ds(0, 32)] = zb[:32]
    pf0[pl.ds(Sp + 32, 32)] = zb[:32]
    pf1[pl.ds(Sp + 32, 32)] = zb[:32]
    pf2[pl.ds(Sp + 32, 32)] = zb[:32]

    mrow = (jax.lax.broadcasted_iota(jnp.int32, (1, _CHUNK), 1) % _WP
            < W).astype(jnp.bfloat16)

    for ci in range(Sp // _CHUNK):
        base = ci * _CHUNK
        xc = jnp.concatenate([xa[:, base:base + _CHUNK], mrow], axis=0)
        h = jax.lax.dot_general(
            xc, we_ref[...],
            (((0,), (0,)), ((), ())),
            preferred_element_type=jnp.float32)
        hbc = jnp.clip(h.astype(jnp.bfloat16),
                       jnp.bfloat16(0.0), jnp.bfloat16(6.0))
        pf0[pl.ds(33 + base, _CHUNK)] = hbc
        pf1[pl.ds(32 + base, _CHUNK)] = hbc
        pf2[pl.ds(31 + base, _CHUNK)] = hbc

    pfs = (pf0, pf1, pf2)
    for ci in range(Sp // _CHUNK):
        base = ci * _CHUNK
        acc = None
        for dh in range(3):
            for dx in range(3):
                slab = pfs[dx][pl.ds(32 * dh + base, _CHUNK)]
                term = slab * wd_ref[3 * dh + dx, :]
                acc = term if acc is None else acc + term
        hdw[pl.ds(base, _CHUNK)] = jnp.clip(acc + bd_ref[...], 0.0, 6.0)

    y = jax.lax.dot_general(
        wp_ref[...], hdw[...],
        (((0,), (1,)), ((), ())),
        preferred_element_type=jnp.float32)
    return (y + bp_ref[...]).astype(jnp.bfloat16) + xa


def _block_body(x_ref, we_ref, wd_ref, bd_ref, wp_ref, bp_ref, o_ref,
                *scratch, C, W, hid, Sp, pair):
    sets = [scratch[4 * i:4 * i + 4] for i in range(_NSETS)]
    args = (we_ref, wd_ref, bd_ref, wp_ref, bp_ref)
    kw = dict(C=C, W=W, hid=hid, Sp=Sp)
    for img in range(pair):
        o_ref[img] = _one_image(x_ref[img], *args, *sets[img % _NSETS], **kw)


def kernel(x_nchw, w_exp, b_exp, w_dw, b_dw, w_proj, b_proj):
    N, C, H, W = x_nchw.shape
    hid = w_exp.shape[1]
    Sp = H * _WP

    xb = x_nchw.astype(jnp.bfloat16)
    xp = jnp.pad(xb, ((0, 0), (0, 0), (0, 0), (0, _WP - W)))
    xa = xp.reshape(N, C, Sp)

    we = jnp.concatenate([w_exp, b_exp], axis=0).astype(jnp.bfloat16)
    wd = w_dw.astype(jnp.bfloat16)
    bd = b_dw.astype(jnp.bfloat16)
    wp = w_proj.astype(jnp.bfloat16)
    bp = jnp.transpose(b_proj)

    pair = math.gcd(N, _PAIR)
    body = functools.partial(_block_body, C=C, W=W, hid=hid, Sp=Sp, pair=pair)

    def full(shape):
        nd = len(shape)
        return pl.BlockSpec(shape, lambda n, nd=nd: (0,) * nd)

    scratch = []
    for _ in range(_NSETS):
        scratch += [pltpu.VMEM((Sp + 64, hid), jnp.bfloat16) for _ in range(3)]
        scratch.append(pltpu.VMEM((Sp, hid), jnp.bfloat16))

    out = pl.pallas_call(
        body,
        out_shape=jax.ShapeDtypeStruct((N, C, Sp), jnp.bfloat16),
        grid=(N // pair,),
        in_specs=[
            pl.BlockSpec((pair, C, Sp), lambda n: (n, 0, 0)),
            full(we.shape),
            full(wd.shape),
            full(bd.shape),
            full(wp.shape),
            full(bp.shape),
        ],
        out_specs=pl.BlockSpec((pair, C, Sp), lambda n: (n, 0, 0)),
        scratch_shapes=scratch,
        compiler_params=pltpu.CompilerParams(
            dimension_semantics=("parallel",)),
    )(xa, we, wd, bd, wp, bp)

    return out.reshape(N, C, H, _WP)[..., :W].astype(jnp.float32)
